# baseline (speedup 1.0000x reference)
import functools

import jax
import jax.numpy as jnp
from jax import lax
from jax.experimental import pallas as pl
from jax.experimental.pallas import tpu as pltpu
from jax.experimental.pallas import tpu_sc as plsc

S = 200
B = 4096
D = 64
O = 2
NW = 32
B_PER_W = B // NW


def _sc_body(x_hbm, table_hbm, pooled_hbm, xchunk_v, buf_a, buf_b,
             pooled_v, sem_a, sem_b):
    wid = lax.axis_index("s") * 2 + lax.axis_index("c")
    base = wid * B_PER_W

    pltpu.sync_copy(x_hbm.at[:, pl.ds(base, B_PER_W)], xchunk_v)

    z = jnp.zeros((16,), jnp.float32)

    def zero(c, carry):
        pooled_v[c, pl.ds(0, 16)] = z
        pooled_v[c, pl.ds(16, 16)] = z
        pooled_v[c, pl.ds(32, 16)] = z
        pooled_v[c, pl.ds(48, 16)] = z
        return carry

    lax.fori_loop(0, B_PER_W, zero, 0, unroll=4)

    def issue(s, buf, sem):
        pltpu.async_copy(table_hbm.at[xchunk_v.at[s]], buf, sem)

    def drain(buf, sem):
        pltpu.make_async_copy(table_hbm.at[xchunk_v.at[0]], buf, sem).wait()

    def accum(buf):
        def body(c, carry):
            plsc.addupdate(pooled_v.at[c, pl.ds(0, 16)],
                           buf[c, pl.ds(0, 16)])
            plsc.addupdate(pooled_v.at[c, pl.ds(16, 16)],
                           buf[c, pl.ds(16, 16)])
            plsc.addupdate(pooled_v.at[c, pl.ds(32, 16)],
                           buf[c, pl.ds(32, 16)])
            plsc.addupdate(pooled_v.at[c, pl.ds(48, 16)],
                           buf[c, pl.ds(48, 16)])
            return carry

        lax.fori_loop(0, B_PER_W, body, 0, unroll=4)

    issue(0, buf_a, sem_a)

    def per_pair(i, carry):
        issue(2 * i + 1, buf_b, sem_b)
        drain(buf_a, sem_a)
        accum(buf_a)

        @pl.when(i < S // 2 - 1)
        def _():
            issue(2 * i + 2, buf_a, sem_a)

        drain(buf_b, sem_b)
        accum(buf_b)
        return carry

    lax.fori_loop(0, S // 2, per_pair, 0)

    pltpu.sync_copy(pooled_v, pooled_hbm.at[pl.ds(base, B_PER_W)])


@jax.jit
def _fast_text(x, emb_table, w_scaled, b_row):
    mesh = plsc.VectorSubcoreMesh(core_axis_name="c", subcore_axis_name="s")
    pooled = functools.partial(
        pl.kernel,
        out_type=jax.ShapeDtypeStruct((B, D), jnp.float32),
        mesh=mesh,
        compiler_params=pltpu.CompilerParams(use_tc_tiling_on_sc=False),
        scratch_types=[
            pltpu.VMEM((S, B_PER_W), jnp.int32),
            pltpu.VMEM((B_PER_W, D), jnp.float32),
            pltpu.VMEM((B_PER_W, D), jnp.float32),
            pltpu.VMEM((B_PER_W, D), jnp.float32),
            pltpu.SemaphoreType.DMA,
            pltpu.SemaphoreType.DMA,
        ],
    )(_sc_body)(x, emb_table)

    def _fc_body(p_ref, w_ref, b_ref, o_ref):
        o_ref[...] = (
            jnp.dot(p_ref[...], w_ref[...],
                    preferred_element_type=jnp.float32)
            + b_ref[...]
        )

    return pl.pallas_call(
        _fc_body,
        out_shape=jax.ShapeDtypeStruct((B, O), jnp.float32),
    )(pooled, w_scaled, b_row)


def kernel(x, emb_table, fc_w, fc_b):
    w_scaled = (fc_w.astype(jnp.float32) / S).T
    b_row = fc_b.astype(jnp.float32)[None, :]
    return _fast_text(x.astype(jnp.int32), emb_table, w_scaled, b_row)


# speedup vs baseline: 3.1263x; 3.1263x over previous
import functools

import jax
import jax.numpy as jnp
from jax import lax
from jax.experimental import pallas as pl
from jax.experimental.pallas import tpu as pltpu
from jax.experimental.pallas import tpu_sc as plsc

S = 200
B = 4096
D = 64
O = 2
V = 1000000
NW = 32
B_PER_W = B // NW
C_BLK = 8192


def _proj_body(tT_ref, w_ref, o0_ref, o1_ref):
    blk = tT_ref[...]
    r = jnp.dot(w_ref[...], blk, preferred_element_type=jnp.float32)
    o0_ref[...] = r[0]
    o1_ref[...] = r[1]


def _sc_body(x_hbm, t0_hbm, t1_hbm, b_hbm, out_hbm,
             xchunk_v, g0a, g1a, g0b, g1b, b_v, out_v, sem_a, sem_b):
    wid = lax.axis_index("s") * 2 + lax.axis_index("c")
    base = wid * B_PER_W

    pltpu.sync_copy(x_hbm.at[:, pl.ds(base, B_PER_W)], xchunk_v)
    pltpu.sync_copy(b_hbm, b_v)

    def issue(s, g0, g1, sem):
        idx = xchunk_v.at[s]
        pltpu.async_copy(t0_hbm.at[idx], g0, sem)
        pltpu.async_copy(t1_hbm.at[idx], g1, sem)

    def drain(g0, g1, sem):
        pltpu.make_async_copy(t0_hbm.at[xchunk_v.at[0]], g0, sem).wait()
        pltpu.make_async_copy(t1_hbm.at[xchunk_v.at[0]], g1, sem).wait()

    def accum(g0, g1, acc):
        new = []
        for k in range(8):
            new.append(acc[k] + g0[pl.ds(16 * k, 16)])
        for k in range(8):
            new.append(acc[8 + k] + g1[pl.ds(16 * k, 16)])
        return tuple(new)

    issue(0, g0a, g1a, sem_a)

    def per_pair(i, acc):
        issue(2 * i + 1, g0b, g1b, sem_b)
        drain(g0a, g1a, sem_a)
        acc = accum(g0a, g1a, acc)

        @pl.when(i < S // 2 - 1)
        def _():
            issue(2 * i + 2, g0a, g1a, sem_a)

        drain(g0b, g1b, sem_b)
        return accum(g0b, g1b, acc)

    z = jnp.zeros((16,), jnp.float32)
    acc = lax.fori_loop(0, S // 2, per_pair, (z,) * 16)

    for k in range(8):
        out_v[0, pl.ds(16 * k, 16)] = acc[k] + b_v[0, :]
        out_v[1, pl.ds(16 * k, 16)] = acc[8 + k] + b_v[1, :]

    pltpu.sync_copy(out_v, out_hbm.at[:, pl.ds(base, B_PER_W)])


@jax.jit
def _fast_text(x, tableT, w_scaled, b_exp):
    n_blk = (V + C_BLK - 1) // C_BLK
    t0, t1 = pl.pallas_call(
        _proj_body,
        grid=(n_blk,),
        in_specs=[
            pl.BlockSpec((D, C_BLK), lambda i: (0, i)),
            pl.BlockSpec((O, D), lambda i: (0, 0)),
        ],
        out_specs=[
            pl.BlockSpec((C_BLK,), lambda i: (i,)),
            pl.BlockSpec((C_BLK,), lambda i: (i,)),
        ],
        out_shape=[
            jax.ShapeDtypeStruct((V,), jnp.float32),
            jax.ShapeDtypeStruct((V,), jnp.float32),
        ],
    )(tableT, w_scaled)

    mesh = plsc.VectorSubcoreMesh(core_axis_name="c", subcore_axis_name="s")
    out = functools.partial(
        pl.kernel,
        out_type=jax.ShapeDtypeStruct((O, B), jnp.float32),
        mesh=mesh,
        compiler_params=pltpu.CompilerParams(use_tc_tiling_on_sc=False),
        scratch_types=[
            pltpu.VMEM((S, B_PER_W), jnp.int32),
            pltpu.VMEM((B_PER_W,), jnp.float32),
            pltpu.VMEM((B_PER_W,), jnp.float32),
            pltpu.VMEM((B_PER_W,), jnp.float32),
            pltpu.VMEM((B_PER_W,), jnp.float32),
            pltpu.VMEM((O, 16), jnp.float32),
            pltpu.VMEM((O, B_PER_W), jnp.float32),
            pltpu.SemaphoreType.DMA,
            pltpu.SemaphoreType.DMA,
        ],
    )(_sc_body)(x, t0, t1, b_exp)
    return out


def kernel(x, emb_table, fc_w, fc_b):
    tableT = emb_table.T
    w_scaled = fc_w.astype(jnp.float32) / S
    b_exp = jnp.broadcast_to(fc_b.astype(jnp.float32)[:, None], (O, 16))
    out = _fast_text(x.astype(jnp.int32), tableT, w_scaled, b_exp)
    return out.T


# speedup vs baseline: 4.4626x; 1.4275x over previous
import functools

import jax
import jax.numpy as jnp
from jax import lax
from jax.experimental import pallas as pl
from jax.experimental.pallas import tpu as pltpu
from jax.experimental.pallas import tpu_sc as plsc

S = 200
B = 4096
D = 64
O = 2
V = 1000000
NW = 32
B_PER_W = B // NW
C_BLK = 16384


def _proj_body(tT_ref, w_ref, o0_ref, o1_ref):
    blk = tT_ref[...]
    r = jnp.dot(w_ref[...], blk, preferred_element_type=jnp.float32)
    o0_ref[...] = r[0]
    o1_ref[...] = r[1]


NBUF = 8


def _sc_body(x_hbm, t0_hbm, t1_hbm, b_hbm, out_hbm,
             xchunk_v, gbuf, b_v, out_v, *sems):
    wid = lax.axis_index("s") * 2 + lax.axis_index("c")
    base = wid * B_PER_W

    pltpu.sync_copy(x_hbm.at[:, pl.ds(base, B_PER_W)], xchunk_v)
    pltpu.sync_copy(b_hbm, b_v)

    def issue(s, k):
        idx = xchunk_v.at[s]
        pltpu.async_copy(t0_hbm.at[idx], gbuf.at[k, 0], sems[k])
        pltpu.async_copy(t1_hbm.at[idx], gbuf.at[k, 1], sems[k])

    def drain(k):
        pltpu.make_async_copy(t0_hbm.at[xchunk_v.at[0]],
                              gbuf.at[k, 0], sems[k]).wait()
        pltpu.make_async_copy(t1_hbm.at[xchunk_v.at[0]],
                              gbuf.at[k, 1], sems[k]).wait()

    def accum(k, acc):
        new = []
        for j in range(8):
            new.append(acc[j] + gbuf[k, 0, pl.ds(16 * j, 16)])
        for j in range(8):
            new.append(acc[8 + j] + gbuf[k, 1, pl.ds(16 * j, 16)])
        return tuple(new)

    for k in range(NBUF):
        issue(k, k)

    def per_group(i, acc):
        s0 = i * NBUF
        for k in range(NBUF):
            drain(k)
            acc = accum(k, acc)

            @pl.when(s0 + k + NBUF < S)
            def _():
                issue(s0 + k + NBUF, k)
        return acc

    z = jnp.zeros((16,), jnp.float32)
    acc = lax.fori_loop(0, S // NBUF, per_group, (z,) * 16)

    for k in range(8):
        out_v[0, pl.ds(16 * k, 16)] = acc[k] + b_v[0, :]
        out_v[1, pl.ds(16 * k, 16)] = acc[8 + k] + b_v[1, :]

    pltpu.sync_copy(out_v, out_hbm.at[:, pl.ds(base, B_PER_W)])


@jax.jit
def _fast_text(x, tableT, w_scaled, b_exp):
    n_blk = (V + C_BLK - 1) // C_BLK
    t0, t1 = pl.pallas_call(
        _proj_body,
        grid=(n_blk,),
        in_specs=[
            pl.BlockSpec((D, C_BLK), lambda i: (0, i)),
            pl.BlockSpec((O, D), lambda i: (0, 0)),
        ],
        out_specs=[
            pl.BlockSpec((C_BLK,), lambda i: (i,)),
            pl.BlockSpec((C_BLK,), lambda i: (i,)),
        ],
        out_shape=[
            jax.ShapeDtypeStruct((V,), jnp.float32),
            jax.ShapeDtypeStruct((V,), jnp.float32),
        ],
    )(tableT, w_scaled)

    mesh = plsc.VectorSubcoreMesh(core_axis_name="c", subcore_axis_name="s")
    out = functools.partial(
        pl.kernel,
        out_type=jax.ShapeDtypeStruct((O, B), jnp.float32),
        mesh=mesh,
        compiler_params=pltpu.CompilerParams(use_tc_tiling_on_sc=False),
        scratch_types=[
            pltpu.VMEM((S, B_PER_W), jnp.int32),
            pltpu.VMEM((NBUF, O, B_PER_W), jnp.float32),
            pltpu.VMEM((O, 16), jnp.float32),
            pltpu.VMEM((O, B_PER_W), jnp.float32),
        ] + [pltpu.SemaphoreType.DMA] * NBUF,
    )(_sc_body)(x, t0, t1, b_exp)
    return out


def kernel(x, emb_table, fc_w, fc_b):
    tableT = emb_table.T
    w_scaled = fc_w.astype(jnp.float32) / S
    b_exp = jnp.broadcast_to(fc_b.astype(jnp.float32)[:, None], (O, 16))
    out = _fast_text(x.astype(jnp.int32), tableT, w_scaled, b_exp)
    return out.T


# speedup vs baseline: 4.6959x; 1.0523x over previous
import functools

import jax
import jax.numpy as jnp
from jax import lax
from jax.experimental import pallas as pl
from jax.experimental.pallas import tpu as pltpu
from jax.experimental.pallas import tpu_sc as plsc

S = 200
B = 4096
D = 64
O = 2
V = 1000000
NW = 32
B_PER_W = B // NW
C_BLK = 32768


def _proj_body(tT_ref, w_ref, o0_ref, o1_ref):
    blk = tT_ref[...]
    r = jnp.dot(w_ref[...], blk, preferred_element_type=jnp.float32)
    o0_ref[...] = r[0]
    o1_ref[...] = r[1]


NBUF = 20


def _sc_body(x_hbm, t0_hbm, t1_hbm, b_hbm, out_hbm,
             xchunk_v, gbuf, b_v, out_v, *sems):
    wid = lax.axis_index("s") * 2 + lax.axis_index("c")
    base = wid * B_PER_W

    pltpu.sync_copy(x_hbm.at[:, pl.ds(base, B_PER_W)], xchunk_v)
    pltpu.sync_copy(b_hbm, b_v)

    def issue(s, k):
        idx = xchunk_v.at[s]
        pltpu.async_copy(t0_hbm.at[idx], gbuf.at[k, 0], sems[k])
        pltpu.async_copy(t1_hbm.at[idx], gbuf.at[k, 1], sems[k])

    def drain(k):
        pltpu.make_async_copy(t0_hbm.at[xchunk_v.at[0]],
                              gbuf.at[k, 0], sems[k]).wait()
        pltpu.make_async_copy(t1_hbm.at[xchunk_v.at[0]],
                              gbuf.at[k, 1], sems[k]).wait()

    def accum(k, acc):
        new = []
        for j in range(8):
            new.append(acc[j] + gbuf[k, 0, pl.ds(16 * j, 16)])
        for j in range(8):
            new.append(acc[8 + j] + gbuf[k, 1, pl.ds(16 * j, 16)])
        return tuple(new)

    for k in range(NBUF):
        issue(k, k)

    def per_group(i, acc):
        s0 = i * NBUF
        for k in range(NBUF):
            drain(k)
            acc = accum(k, acc)

            @pl.when(s0 + k + NBUF < S)
            def _():
                issue(s0 + k + NBUF, k)
        return acc

    z = jnp.zeros((16,), jnp.float32)
    acc = lax.fori_loop(0, S // NBUF, per_group, (z,) * 16)

    for k in range(8):
        out_v[0, pl.ds(16 * k, 16)] = acc[k] + b_v[0, :]
        out_v[1, pl.ds(16 * k, 16)] = acc[8 + k] + b_v[1, :]

    pltpu.sync_copy(out_v, out_hbm.at[:, pl.ds(base, B_PER_W)])


@jax.jit
def _fast_text(x, tableT, w_scaled, b_exp):
    n_blk = (V + C_BLK - 1) // C_BLK
    t0, t1 = pl.pallas_call(
        _proj_body,
        grid=(n_blk,),
        in_specs=[
            pl.BlockSpec((D, C_BLK), lambda i: (0, i)),
            pl.BlockSpec((O, D), lambda i: (0, 0)),
        ],
        out_specs=[
            pl.BlockSpec((C_BLK,), lambda i: (i,)),
            pl.BlockSpec((C_BLK,), lambda i: (i,)),
        ],
        out_shape=[
            jax.ShapeDtypeStruct((V,), jnp.float32),
            jax.ShapeDtypeStruct((V,), jnp.float32),
        ],
    )(tableT, w_scaled)

    mesh = plsc.VectorSubcoreMesh(core_axis_name="c", subcore_axis_name="s")
    out = functools.partial(
        pl.kernel,
        out_type=jax.ShapeDtypeStruct((O, B), jnp.float32),
        mesh=mesh,
        compiler_params=pltpu.CompilerParams(use_tc_tiling_on_sc=False),
        scratch_types=[
            pltpu.VMEM((S, B_PER_W), jnp.int32),
            pltpu.VMEM((NBUF, O, B_PER_W), jnp.float32),
            pltpu.VMEM((O, 16), jnp.float32),
            pltpu.VMEM((O, B_PER_W), jnp.float32),
        ] + [pltpu.SemaphoreType.DMA] * NBUF,
    )(_sc_body)(x, t0, t1, b_exp)
    return out


def kernel(x, emb_table, fc_w, fc_b):
    tableT = emb_table.T
    w_scaled = fc_w.astype(jnp.float32) / S
    b_exp = jnp.broadcast_to(fc_b.astype(jnp.float32)[:, None], (O, 16))
    out = _fast_text(x.astype(jnp.int32), tableT, w_scaled, b_exp)
    return out.T


# speedup vs baseline: 5.7142x; 1.2169x over previous
import functools

import jax
import jax.numpy as jnp
from jax import lax
from jax.experimental import pallas as pl
from jax.experimental.pallas import tpu as pltpu
from jax.experimental.pallas import tpu_sc as plsc

S = 200
B = 4096
D = 64
O = 2
V = 1000000
NW = 32
B_PER_W = B // NW
C_BLK = 32768
NBUF = 20


def _rtne_hi16(u):
    return (u + jnp.uint32(0x7FFF) + ((u >> 16) & jnp.uint32(1))) \
        & jnp.uint32(0xFFFF0000)


def _proj_body(tT_ref, w_ref, o_ref):
    blk = tT_ref[...]
    r = jnp.dot(w_ref[...], blk, preferred_element_type=jnp.float32)
    u0 = lax.bitcast_convert_type(r[0], jnp.uint32)
    u1 = lax.bitcast_convert_type(r[1], jnp.uint32)
    packed = _rtne_hi16(u0) | (_rtne_hi16(u1) >> 16)
    o_ref[...] = lax.bitcast_convert_type(packed, jnp.int32)


def _sc_body(x_hbm, t01_hbm, b_hbm, out_hbm,
             xchunk_v, gbuf, b_v, out_v, *sems):
    wid = lax.axis_index("s") * 2 + lax.axis_index("c")
    base = wid * B_PER_W

    pltpu.sync_copy(x_hbm.at[:, pl.ds(base, B_PER_W)], xchunk_v)
    pltpu.sync_copy(b_hbm, b_v)

    def issue(s, k):
        pltpu.async_copy(t01_hbm.at[xchunk_v.at[s]], gbuf.at[k], sems[k])

    def drain(k):
        pltpu.make_async_copy(t01_hbm.at[xchunk_v.at[0]],
                              gbuf.at[k], sems[k]).wait()

    hi_mask = jnp.full((16,), -65536, jnp.int32)

    def accum(k, acc):
        new = list(acc)
        for j in range(8):
            v = gbuf[k, pl.ds(16 * j, 16)]
            a0 = lax.bitcast_convert_type(v & hi_mask, jnp.float32)
            a1 = lax.bitcast_convert_type(v << 16, jnp.float32)
            new[j] = acc[j] + a0
            new[8 + j] = acc[8 + j] + a1
        return tuple(new)

    for k in range(NBUF):
        issue(k, k)

    def per_group(i, acc):
        s0 = i * NBUF
        for k in range(NBUF):
            drain(k)
            acc = accum(k, acc)

            @pl.when(s0 + k + NBUF < S)
            def _():
                issue(s0 + k + NBUF, k)
        return acc

    z = jnp.zeros((16,), jnp.float32)
    acc = lax.fori_loop(0, S // NBUF, per_group, (z,) * 16)

    for k in range(8):
        out_v[0, pl.ds(16 * k, 16)] = acc[k] + b_v[0, :]
        out_v[1, pl.ds(16 * k, 16)] = acc[8 + k] + b_v[1, :]

    pltpu.sync_copy(out_v, out_hbm.at[:, pl.ds(base, B_PER_W)])


@jax.jit
def _fast_text(x, tableT, w_scaled, b_exp):
    n_blk = (V + C_BLK - 1) // C_BLK
    t01 = pl.pallas_call(
        _proj_body,
        grid=(n_blk,),
        in_specs=[
            pl.BlockSpec((D, C_BLK), lambda i: (0, i)),
            pl.BlockSpec((O, D), lambda i: (0, 0)),
        ],
        out_specs=pl.BlockSpec((C_BLK,), lambda i: (i,)),
        out_shape=jax.ShapeDtypeStruct((V,), jnp.int32),
    )(tableT, w_scaled)

    mesh = plsc.VectorSubcoreMesh(core_axis_name="c", subcore_axis_name="s")
    out = functools.partial(
        pl.kernel,
        out_type=jax.ShapeDtypeStruct((O, B), jnp.float32),
        mesh=mesh,
        compiler_params=pltpu.CompilerParams(use_tc_tiling_on_sc=False),
        scratch_types=[
            pltpu.VMEM((S, B_PER_W), jnp.int32),
            pltpu.VMEM((NBUF, B_PER_W), jnp.int32),
            pltpu.VMEM((O, 16), jnp.float32),
            pltpu.VMEM((O, B_PER_W), jnp.float32),
        ] + [pltpu.SemaphoreType.DMA] * NBUF,
    )(_sc_body)(x, t01, b_exp)
    return out


def kernel(x, emb_table, fc_w, fc_b):
    tableT = emb_table.T
    w_scaled = fc_w.astype(jnp.float32) / S
    b_exp = jnp.broadcast_to(fc_b.astype(jnp.float32)[:, None], (O, 16))
    out = _fast_text(x.astype(jnp.int32), tableT, w_scaled, b_exp)
    return out.T


# speedup vs baseline: 6.6669x; 1.1667x over previous
import functools

import jax
import jax.numpy as jnp
from jax import lax
from jax.experimental import pallas as pl
from jax.experimental.pallas import tpu as pltpu
from jax.experimental.pallas import tpu_sc as plsc

S = 200
B = 4096
D = 64
O = 2
V = 1000000
NW = 32
B_PER_W = B // NW
C_BLK = 32768
NBUF = 20


def _rtne_hi16(u):
    return (u + jnp.uint32(0x7FFF) + ((u >> 16) & jnp.uint32(1))) \
        & jnp.uint32(0xFFFF0000)


def _proj_body(tT_ref, w_ref, o_ref):
    blk = tT_ref[...]
    r = jnp.dot(w_ref[...], blk, preferred_element_type=jnp.float32)
    u0 = lax.bitcast_convert_type(r[0], jnp.uint32)
    u1 = lax.bitcast_convert_type(r[1], jnp.uint32)
    packed = _rtne_hi16(u0) | (_rtne_hi16(u1) >> 16)
    o_ref[...] = lax.bitcast_convert_type(packed, jnp.int32)


CHUNK = 62496


def _sc_body(x_hbm, t01_hbm, b_hbm, out_hbm,
             xchunk_v, gbuf, b_v, out_v, t01_sh, *sems):
    sid = lax.axis_index("s")
    wid = sid * 2 + lax.axis_index("c")
    base = wid * B_PER_W

    off = sid * CHUNK

    @pl.when(sid < 15)
    def _():
        pltpu.sync_copy(t01_hbm.at[pl.ds(off, CHUNK)],
                        t01_sh.at[pl.ds(off, CHUNK)])

    @pl.when(sid == 15)
    def _():
        pltpu.sync_copy(t01_hbm.at[pl.ds(15 * CHUNK, V - 15 * CHUNK)],
                        t01_sh.at[pl.ds(15 * CHUNK, V - 15 * CHUNK)])

    pltpu.sync_copy(x_hbm.at[:, pl.ds(base, B_PER_W)], xchunk_v)
    pltpu.sync_copy(b_hbm, b_v)
    plsc.subcore_barrier()

    def issue(s, k):
        pltpu.async_copy(t01_sh.at[xchunk_v.at[s]], gbuf.at[k], sems[k])

    def drain(k):
        pltpu.make_async_copy(t01_sh.at[xchunk_v.at[0]],
                              gbuf.at[k], sems[k]).wait()

    hi_mask = jnp.full((16,), -65536, jnp.int32)

    def accum(k, acc):
        new = list(acc)
        for j in range(8):
            v = gbuf[k, pl.ds(16 * j, 16)]
            a0 = lax.bitcast_convert_type(v & hi_mask, jnp.float32)
            a1 = lax.bitcast_convert_type(v << 16, jnp.float32)
            new[j] = acc[j] + a0
            new[8 + j] = acc[8 + j] + a1
        return tuple(new)

    for k in range(NBUF):
        issue(k, k)

    def per_group(i, acc):
        s0 = i * NBUF
        for k in range(NBUF):
            drain(k)
            acc = accum(k, acc)

            @pl.when(s0 + k + NBUF < S)
            def _():
                issue(s0 + k + NBUF, k)
        return acc

    z = jnp.zeros((16,), jnp.float32)
    acc = lax.fori_loop(0, S // NBUF, per_group, (z,) * 16)

    for k in range(8):
        out_v[0, pl.ds(16 * k, 16)] = acc[k] + b_v[0, :]
        out_v[1, pl.ds(16 * k, 16)] = acc[8 + k] + b_v[1, :]

    pltpu.sync_copy(out_v, out_hbm.at[:, pl.ds(base, B_PER_W)])


@jax.jit
def _fast_text(x, tableT, w_scaled, b_exp):
    n_blk = (V + C_BLK - 1) // C_BLK
    t01 = pl.pallas_call(
        _proj_body,
        grid=(n_blk,),
        in_specs=[
            pl.BlockSpec((D, C_BLK), lambda i: (0, i)),
            pl.BlockSpec((O, D), lambda i: (0, 0)),
        ],
        out_specs=pl.BlockSpec((C_BLK,), lambda i: (i,)),
        out_shape=jax.ShapeDtypeStruct((V,), jnp.int32),
    )(tableT, w_scaled)

    mesh = plsc.VectorSubcoreMesh(core_axis_name="c", subcore_axis_name="s")
    out = functools.partial(
        pl.kernel,
        out_type=jax.ShapeDtypeStruct((O, B), jnp.float32),
        mesh=mesh,
        compiler_params=pltpu.CompilerParams(use_tc_tiling_on_sc=False),
        scratch_types=[
            pltpu.VMEM((S, B_PER_W), jnp.int32),
            pltpu.VMEM((NBUF, B_PER_W), jnp.int32),
            pltpu.VMEM((O, 16), jnp.float32),
            pltpu.VMEM((O, B_PER_W), jnp.float32),
            pltpu.VMEM_SHARED((V,), jnp.int32),
        ] + [pltpu.SemaphoreType.DMA] * NBUF,
    )(_sc_body)(x, t01, b_exp)
    return out


def kernel(x, emb_table, fc_w, fc_b):
    tableT = emb_table.T
    w_scaled = fc_w.astype(jnp.float32) / S
    b_exp = jnp.broadcast_to(fc_b.astype(jnp.float32)[:, None], (O, 16))
    out = _fast_text(x.astype(jnp.int32), tableT, w_scaled, b_exp)
    return out.T
